# baseline (speedup 1.0000x reference)
import functools

import jax
import jax.numpy as jnp
from jax import lax
from jax.experimental import pallas as pl
from jax.experimental.pallas import tpu as pltpu
from jax.experimental.pallas import tpu_sc as plsc

_NUM_ITEMS = 1_000_000
_STAGE = 65_536
_TABLE_HBM = _STAGE * 16
_ZERO_SLOT = _TABLE_HBM
_TABLE_SP = _TABLE_HBM + 16
_NC, _NS = 2, 16
_NW = _NC * _NS
_BATCH, _HIST = 16384, 200
_B_TOTAL = _BATCH * _HIST
_PER_W = _B_TOTAL // _NW
_CHUNK = 12_800
_HALF = _CHUNK // 2
_N_CHUNKS = _PER_W // _CHUNK

_SIG_BLK = 65_536
_SIG_GRID = (_TABLE_HBM + _SIG_BLK - 1) // _SIG_BLK


def _sigmoid_body(x_ref, o_ref):
    o_ref[...] = jax.nn.sigmoid(x_ref[...])


def _build_table(logits):
    return pl.pallas_call(
        _sigmoid_body,
        out_shape=jax.ShapeDtypeStruct((_TABLE_HBM,), jnp.float32),
        grid=(_SIG_GRID,),
        in_specs=[pl.BlockSpec((_SIG_BLK,), lambda i: (i,))],
        out_specs=pl.BlockSpec((_SIG_BLK,), lambda i: (i,)),
    )(logits)


def _remap_body(i_ref, o_ref):
    i = i_ref[...]
    o_ref[...] = jnp.where(i == 0, jnp.int32(_ZERO_SLOT), i - 1)


def _remap_indices(idx2d):
    return pl.pallas_call(
        _remap_body,
        out_shape=jax.ShapeDtypeStruct((_BATCH, _HIST), jnp.int32),
        grid=(8,),
        in_specs=[pl.BlockSpec((_BATCH // 8, _HIST), lambda i: (i, 0))],
        out_specs=pl.BlockSpec((_BATCH // 8, _HIST), lambda i: (i, 0)),
    )(idx2d)


@functools.partial(
    pl.kernel,
    out_type=jax.ShapeDtypeStruct((_B_TOTAL,), jnp.float32),
    mesh=plsc.VectorSubcoreMesh(core_axis_name="c", subcore_axis_name="s"),
    scratch_types=[
        pltpu.VMEM((_CHUNK,), jnp.int32),
        pltpu.VMEM((_CHUNK,), jnp.int32),
        pltpu.VMEM((_CHUNK,), jnp.float32),
        pltpu.VMEM((_CHUNK,), jnp.float32),
        pltpu.VMEM((16,), jnp.float32),
        pltpu.VMEM_SHARED((_TABLE_SP,), jnp.float32),
        pltpu.SemaphoreType.DMA,
        pltpu.SemaphoreType.DMA,
        pltpu.SemaphoreType.DMA,
        pltpu.SemaphoreType.DMA,
    ],
)
def _gather_kernel(table_hbm, idx_hbm, out_hbm,
                   idx_v0, idx_v1, val_v0, val_v1, zero_v, table_sp,
                   lsem, gsem, gsem2, ssem):
    sid = lax.axis_index("s")
    wid = sid * _NC + lax.axis_index("c")
    base = wid * _PER_W

    soff = sid * _STAGE
    stage = pltpu.async_copy(table_hbm.at[pl.ds(soff, _STAGE)],
                             table_sp.at[pl.ds(soff, _STAGE)], gsem)

    idx_bufs = [idx_v0, idx_v1]
    val_bufs = [val_v0, val_v1]
    loads = [None] * _N_CHUNKS
    loads[0] = pltpu.async_copy(idx_hbm.at[pl.ds(base, _CHUNK)], idx_v0, lsem)

    @pl.when(sid == 0)
    def _zero_slot():
        zero_v[...] = jnp.zeros((16,), jnp.float32)
        pltpu.sync_copy(zero_v, table_sp.at[pl.ds(_ZERO_SLOT, 16)])

    stage.wait()
    plsc.subcore_barrier()

    stores = [None, None]
    for i in range(_N_CHUNKS):
        cur = i % 2
        loads[i].wait()
        if i + 1 < _N_CHUNKS:
            loads[i + 1] = pltpu.async_copy(
                idx_hbm.at[pl.ds(base + (i + 1) * _CHUNK, _CHUNK)],
                idx_bufs[(i + 1) % 2], lsem)
        if stores[cur] is not None:
            stores[cur].wait()
        g0 = pltpu.async_copy(table_sp.at[idx_bufs[cur].at[pl.ds(0, _HALF)]],
                              val_bufs[cur].at[pl.ds(0, _HALF)], gsem)
        g1 = pltpu.async_copy(table_sp.at[idx_bufs[cur].at[pl.ds(_HALF, _HALF)]],
                              val_bufs[cur].at[pl.ds(_HALF, _HALF)], gsem2)
        g0.wait()
        g1.wait()
        stores[cur] = pltpu.async_copy(
            val_bufs[cur], out_hbm.at[pl.ds(base + i * _CHUNK, _CHUNK)], ssem)
    for h in stores:
        if h is not None:
            h.wait()


def kernel(indices, item_rating_logits):
    idx2 = _remap_indices(indices.astype(jnp.int32)).reshape(-1)
    table = _build_table(item_rating_logits.astype(jnp.float32))
    return _gather_kernel(table, idx2).reshape(_BATCH, _HIST)


# speedup vs baseline: 1.1590x; 1.1590x over previous
import functools

import jax
import jax.numpy as jnp
from jax import lax
from jax.experimental import pallas as pl
from jax.experimental.pallas import tpu as pltpu
from jax.experimental.pallas import tpu_sc as plsc

_NUM_ITEMS = 1_000_000
_STAGE = 65_536
_TABLE_HBM = _STAGE * 16
_ZERO_SLOT = _TABLE_HBM
_TABLE_SP = _TABLE_HBM + 16
_NC, _NS = 2, 16
_NW = _NC * _NS
_BATCH, _HIST = 16384, 200
_B_TOTAL = _BATCH * _HIST
_PER_W = _B_TOTAL // _NW
_CHUNK = 12_800
_HALF = _CHUNK // 2
_N_CHUNKS = _PER_W // _CHUNK

_SIG_BLK = 65_536
_SIG_GRID = (_TABLE_HBM + _SIG_BLK - 1) // _SIG_BLK


def _sigmoid_body(x_ref, o_ref):
    o_ref[...] = jax.nn.sigmoid(x_ref[...])


def _build_table(logits):
    return pl.pallas_call(
        _sigmoid_body,
        out_shape=jax.ShapeDtypeStruct((_TABLE_HBM,), jnp.float32),
        grid=(_SIG_GRID,),
        in_specs=[pl.BlockSpec((_SIG_BLK,), lambda i: (i,))],
        out_specs=pl.BlockSpec((_SIG_BLK,), lambda i: (i,)),
    )(logits)


def _remap_chunk(idx_ref):
    @plsc.parallel_loop(0, _CHUNK, step=16, unroll=8)
    def _r(off):
        iv = idx_ref[pl.ds(off, 16)]
        idx_ref[pl.ds(off, 16)] = jnp.where(
            iv == 0, jnp.int32(_ZERO_SLOT), iv - 1)


@functools.partial(
    pl.kernel,
    out_type=jax.ShapeDtypeStruct((_B_TOTAL,), jnp.float32),
    mesh=plsc.VectorSubcoreMesh(core_axis_name="c", subcore_axis_name="s"),
    scratch_types=[
        pltpu.VMEM((_CHUNK,), jnp.int32),
        pltpu.VMEM((_CHUNK,), jnp.int32),
        pltpu.VMEM((_CHUNK,), jnp.float32),
        pltpu.VMEM((_CHUNK,), jnp.float32),
        pltpu.VMEM((16,), jnp.float32),
        pltpu.VMEM_SHARED((_TABLE_SP,), jnp.float32),
        pltpu.SemaphoreType.DMA,
        pltpu.SemaphoreType.DMA,
        pltpu.SemaphoreType.DMA,
    ],
)
def _gather_kernel(table_hbm, idx_hbm, out_hbm,
                   idx_v0, idx_v1, val_v0, val_v1, zero_v, table_sp,
                   lsem, gsem, ssem):
    sid = lax.axis_index("s")
    wid = sid * _NC + lax.axis_index("c")
    base = wid * _PER_W

    soff = sid * _STAGE
    stage = pltpu.async_copy(table_hbm.at[pl.ds(soff, _STAGE)],
                             table_sp.at[pl.ds(soff, _STAGE)], gsem)

    idx_bufs = [idx_v0, idx_v1]
    val_bufs = [val_v0, val_v1]
    loads = [None] * _N_CHUNKS
    loads[0] = pltpu.async_copy(idx_hbm.at[pl.ds(base, _CHUNK)], idx_v0, lsem)

    @pl.when(sid == 0)
    def _zero_slot():
        zero_v[...] = jnp.zeros((16,), jnp.float32)
        pltpu.sync_copy(zero_v, table_sp.at[pl.ds(_ZERO_SLOT, 16)])

    loads[0].wait()
    _remap_chunk(idx_v0)

    stage.wait()
    plsc.subcore_barrier()

    stores = [None, None]
    for i in range(_N_CHUNKS):
        cur = i % 2
        nxt = (i + 1) % 2
        if stores[cur] is not None:
            stores[cur].wait()
        g = pltpu.async_copy(table_sp.at[idx_bufs[cur]], val_bufs[cur], gsem)
        if i + 1 < _N_CHUNKS:
            loads[i + 1] = pltpu.async_copy(
                idx_hbm.at[pl.ds(base + (i + 1) * _CHUNK, _CHUNK)],
                idx_bufs[nxt], lsem)
            loads[i + 1].wait()
            _remap_chunk(idx_bufs[nxt])
        g.wait()
        stores[cur] = pltpu.async_copy(
            val_bufs[cur], out_hbm.at[pl.ds(base + i * _CHUNK, _CHUNK)], ssem)
    for h in stores:
        if h is not None:
            h.wait()


def kernel(indices, item_rating_logits):
    idx = indices.astype(jnp.int32).reshape(-1)
    table = _build_table(item_rating_logits.astype(jnp.float32))
    return _gather_kernel(table, idx).reshape(_BATCH, _HIST)
